# baseline (speedup 1.0000x reference)
import functools

import jax
import jax.numpy as jnp
from jax import lax
from jax.experimental import pallas as pl
from jax.experimental.pallas import tpu as pltpu
from jax.experimental.pallas import tpu_sc as plsc

_NC = 2
_NS = 16
_NW = _NC * _NS
_L = 16


def _make_sc_assign(E):
    per = E // _NW
    C = 5000 if per % 5000 == 0 else per
    n_chunks = per // C
    n_vec = (C + _L - 1) // _L
    c_pad = n_vec * _L

    mesh = plsc.VectorSubcoreMesh(core_axis_name="c", subcore_axis_name="s")

    idx_buf = pltpu.VMEM((c_pad,), jnp.int32)
    data_buf = pltpu.VMEM((c_pad,), jnp.float32)

    @functools.partial(
        pl.kernel,
        mesh=mesh,
        out_type=jax.ShapeDtypeStruct((E,), jnp.int32),
        scratch_types=[
            idx_buf, idx_buf,
            idx_buf, idx_buf, idx_buf,
            idx_buf, idx_buf, idx_buf,
            data_buf, data_buf, data_buf,
            data_buf, data_buf, data_buf,
            pltpu.VMEM((c_pad,), jnp.int32),
            pltpu.VMEM((6 * _L,), jnp.float32),
            pltpu.SemaphoreType.DMA,
            pltpu.SemaphoreType.DMA,
        ],
    )
    def sc_assign(ref_hbm, query_hbm, eref_hbm, equery_hbm, kp_hbm, out_hbm,
                  er_v, eq_v, rix, riy, riz, qix, qiy, qiz,
                  rx_v, ry_v, rz_v, qx_v, qy_v, qz_v,
                  out_v, kp_v, sem_r, sem_q):
        wid = (lax.axis_index("s").astype(jnp.int32) * jnp.int32(_NC)
               + lax.axis_index("c").astype(jnp.int32))
        tile_base = wid * jnp.int32(per)

        pltpu.sync_copy(kp_hbm, kp_v)
        hxv = kp_v[pl.ds(0, _L)]
        hyv = kp_v[pl.ds(_L, _L)]
        hzv = kp_v[pl.ds(2 * _L, _L)]
        nhxv = kp_v[pl.ds(3 * _L, _L)]
        nhyv = kp_v[pl.ds(4 * _L, _L)]
        nhzv = kp_v[pl.ds(5 * _L, _L)]
        nine = jnp.full((_L,), 9, jnp.int32)
        three = jnp.full((_L,), 3, jnp.int32)
        one_i = jnp.full((_L,), 1, jnp.int32)
        zero = jnp.full((_L,), 0, jnp.int32)
        two = jnp.full((_L,), 2, jnp.int32)
        four = jnp.full((_L,), 4, jnp.int32)

        for j in range(n_chunks):
            base_e = tile_base + jnp.int32(j * C)
            pltpu.sync_copy(eref_hbm.at[pl.ds(base_e, C)],
                            er_v.at[pl.ds(0, C)])
            pltpu.sync_copy(equery_hbm.at[pl.ds(base_e, C)],
                            eq_v.at[pl.ds(0, C)])

            def build(i, carry):
                sl = pl.ds(i * jnp.int32(_L), _L)
                r4 = er_v[sl] * four
                q4 = eq_v[sl] * four
                rix[sl] = r4 + one_i
                riy[sl] = r4 + two
                riz[sl] = r4 + three
                qix[sl] = q4 + one_i
                qiy[sl] = q4 + two
                qiz[sl] = q4 + three
                return carry

            lax.fori_loop(jnp.int32(0), jnp.int32(n_vec), build, 0,
                          unroll=False)

            cps = [
                pltpu.async_copy(ref_hbm.at[rix.at[pl.ds(0, C)]],
                                 rx_v.at[pl.ds(0, C)], sem_r),
                pltpu.async_copy(ref_hbm.at[riy.at[pl.ds(0, C)]],
                                 ry_v.at[pl.ds(0, C)], sem_r),
                pltpu.async_copy(ref_hbm.at[riz.at[pl.ds(0, C)]],
                                 rz_v.at[pl.ds(0, C)], sem_r),
                pltpu.async_copy(query_hbm.at[qix.at[pl.ds(0, C)]],
                                 qx_v.at[pl.ds(0, C)], sem_q),
                pltpu.async_copy(query_hbm.at[qiy.at[pl.ds(0, C)]],
                                 qy_v.at[pl.ds(0, C)], sem_q),
                pltpu.async_copy(query_hbm.at[qiz.at[pl.ds(0, C)]],
                                 qz_v.at[pl.ds(0, C)], sem_q),
            ]
            for cp in cps:
                cp.wait()

            def body(i, carry):
                sl = pl.ds(i * jnp.int32(_L), _L)
                tx = rx_v[sl] - qx_v[sl]
                ty = ry_v[sl] - qy_v[sl]
                tz = rz_v[sl] - qz_v[sl]
                out_v[sl] = (jnp.where(tx > nhxv, nine, zero)
                             + jnp.where(tx > hxv, nine, zero)
                             + jnp.where(ty > nhyv, three, zero)
                             + jnp.where(ty > hyv, three, zero)
                             + jnp.where(tz > nhzv, one_i, zero)
                             + jnp.where(tz > hzv, one_i, zero))
                return carry

            lax.fori_loop(jnp.int32(0), jnp.int32(n_vec), body, 0,
                          unroll=False)
            pltpu.sync_copy(out_v.at[pl.ds(0, C)],
                            out_hbm.at[pl.ds(base_e, C)])

    return sc_assign


def kernel(ref_bxyz, query_bxyz, e_ref, e_query, kernel_pos):
    E = e_ref.shape[0]
    er = e_ref.astype(jnp.int32)
    eq = e_query.astype(jnp.int32)
    ref_flat = ref_bxyz.reshape(-1)
    query_flat = query_bxyz.reshape(-1)
    h = kernel_pos[26, :].astype(jnp.float32) * jnp.float32(0.5)
    kp_pad = jnp.concatenate([jnp.repeat(h, _L), jnp.repeat(-h, _L)])
    out32 = _make_sc_assign(E)(ref_flat, query_flat, er, eq, kp_pad)
    return out32.astype(jnp.int64)


# speedup vs baseline: 1.9835x; 1.9835x over previous
import functools

import jax
import jax.numpy as jnp
from jax import lax
from jax.experimental import pallas as pl
from jax.experimental.pallas import tpu as pltpu
from jax.experimental.pallas import tpu_sc as plsc

_NC = 2
_NS = 16
_NW = _NC * _NS
_L = 16


def _make_sc_assign(E, N):
    per = E // _NW
    C = 5000 if per % 5000 == 0 else per
    n_chunks = per // C
    n_vec = (C + _L - 1) // _L
    c_pad = n_vec * _L

    mesh = plsc.VectorSubcoreMesh(core_axis_name="c", subcore_axis_name="s")

    idx_buf = pltpu.VMEM((c_pad,), jnp.int32)
    col_buf = pltpu.VMEM((N,), jnp.float32)

    @functools.partial(
        pl.kernel,
        mesh=mesh,
        out_type=jax.ShapeDtypeStruct((E,), jnp.int32),
        compiler_params=pltpu.CompilerParams(needs_layout_passes=False),
        scratch_types=[
            idx_buf, idx_buf,
            col_buf, col_buf,
            pltpu.VMEM((c_pad,), jnp.int32),
            pltpu.VMEM((6 * _L,), jnp.float32),
            pltpu.SemaphoreType.DMA,
            pltpu.SemaphoreType.DMA,
        ],
    )
    def sc_assign(rx_hbm, ry_hbm, rz_hbm, qx_hbm, qy_hbm, qz_hbm,
                  eref_hbm, equery_hbm, kp_hbm, out_hbm,
                  er_v, eq_v, rcol, qcol, out_v, kp_v, sem_r, sem_q):
        wid = (lax.axis_index("s").astype(jnp.int32) * jnp.int32(_NC)
               + lax.axis_index("c").astype(jnp.int32))
        tile_base = wid * jnp.int32(per)

        pltpu.sync_copy(kp_hbm, kp_v)
        zero = jnp.full((_L,), 0, jnp.int32)
        if c_pad > C:
            er_v[pl.ds(jnp.int32(c_pad - _L), _L)] = zero
            eq_v[pl.ds(jnp.int32(c_pad - _L), _L)] = zero

        axes = [
            (0, 9, rx_hbm, qx_hbm),
            (1, 3, ry_hbm, qy_hbm),
            (2, 1, rz_hbm, qz_hbm),
        ]
        for a, w, rcol_hbm, qcol_hbm in axes:
            cp_r = pltpu.async_copy(rcol_hbm, rcol, sem_r)
            cp_q = pltpu.async_copy(qcol_hbm, qcol, sem_q)
            cp_r.wait()
            cp_q.wait()
            hv = kp_v[pl.ds(a * _L, _L)]
            nhv = kp_v[pl.ds((3 + a) * _L, _L)]
            wv = jnp.full((_L,), w, jnp.int32)

            for j in range(n_chunks):
                base_e = tile_base + jnp.int32(j * C)
                pltpu.sync_copy(eref_hbm.at[pl.ds(base_e, C)],
                                er_v.at[pl.ds(0, C)])
                pltpu.sync_copy(equery_hbm.at[pl.ds(base_e, C)],
                                eq_v.at[pl.ds(0, C)])
                if a > 0:
                    pltpu.sync_copy(out_hbm.at[pl.ds(base_e, C)],
                                    out_v.at[pl.ds(0, C)])

                def body(i, carry):
                    sl = pl.ds(i * jnp.int32(_L), _L)
                    t = (plsc.load_gather(rcol, [er_v[sl]])
                         - plsc.load_gather(qcol, [eq_v[sl]]))
                    contrib = (jnp.where(t > nhv, wv, zero)
                               + jnp.where(t > hv, wv, zero))
                    if a > 0:
                        contrib = contrib + out_v[sl]
                    out_v[sl] = contrib
                    return carry

                lax.fori_loop(jnp.int32(0), jnp.int32(n_vec), body, 0,
                              unroll=False)
                pltpu.sync_copy(out_v.at[pl.ds(0, C)],
                                out_hbm.at[pl.ds(base_e, C)])

    return sc_assign


def kernel(ref_bxyz, query_bxyz, e_ref, e_query, kernel_pos):
    E = e_ref.shape[0]
    N = ref_bxyz.shape[0]
    er = e_ref.astype(jnp.int32)
    eq = e_query.astype(jnp.int32)
    rx, ry, rz = ref_bxyz[:, 1], ref_bxyz[:, 2], ref_bxyz[:, 3]
    qx, qy, qz = query_bxyz[:, 1], query_bxyz[:, 2], query_bxyz[:, 3]
    h = kernel_pos[26, :].astype(jnp.float32) * jnp.float32(0.5)
    kp_pad = jnp.concatenate([jnp.repeat(h, _L), jnp.repeat(-h, _L)])
    out32 = _make_sc_assign(E, N)(rx, ry, rz, qx, qy, qz, er, eq, kp_pad)
    return out32.astype(jnp.int64)
